# Initial kernel scaffold; baseline (speedup 1.0000x reference)
import jax
import jax.numpy as jnp
from jax.experimental import pallas as pl


def kernel(embedding, pos_table):
    raise NotImplementedError("write your pallas kernel here")


# speedup vs baseline: 2.1927x; 2.1927x over previous
import jax
import jax.numpy as jnp
from jax.experimental import pallas as pl


_S_BLK = 256


def _pe_add_kernel(emb_ref, pos_ref, out_ref):
    out_ref[...] = emb_ref[...] + pos_ref[...][:, None, :]


def kernel(embedding, pos_table):
    seq_len, batch, d_model = embedding.shape
    grid = (seq_len // _S_BLK,)
    return pl.pallas_call(
        _pe_add_kernel,
        grid=grid,
        in_specs=[
            pl.BlockSpec((_S_BLK, batch, d_model), lambda i: (i, 0, 0)),
            pl.BlockSpec((_S_BLK, d_model), lambda i: (i, 0)),
        ],
        out_specs=pl.BlockSpec((_S_BLK, batch, d_model), lambda i: (i, 0, 0)),
        out_shape=jax.ShapeDtypeStruct(embedding.shape, embedding.dtype),
    )(embedding, pos_table)
